# baseline (speedup 1.0000x reference)
import functools

import jax
import jax.numpy as jnp
from jax import lax
from jax.experimental import pallas as pl
from jax.experimental.pallas import tpu as pltpu
from jax.experimental.pallas import tpu_sc as plsc

_B = 4096
_L = 50
_HID = 128

_NC = 2
_NS = 16
_NW = _NC * _NS
_CB = 2
_CHUNK = _CB * _L
_NB = 4
_K = 2

_mesh = plsc.VectorSubcoreMesh(core_axis_name="c", subcore_axis_name="s")


@functools.lru_cache(maxsize=None)
def _make_gather(nbatch):
    bpw = nbatch // _NW
    nchunk = bpw // _CB
    ngrp = nchunk // _NB

    @functools.partial(
        pl.kernel,
        mesh=_mesh,
        out_type=jax.ShapeDtypeStruct((nbatch, _L, _HID), jnp.float32),
        scratch_types=[
            pltpu.VMEM((nchunk, _CHUNK), jnp.int32),
            pltpu.VMEM((_NB, _CHUNK, _HID), jnp.float32),
            pltpu.SemaphoreType.DMA((_NB,)),
            pltpu.SemaphoreType.DMA((_NB,)),
        ],
    )
    def gather_kernel(src_hbm, table_hbm, out_hbm, idx_v, rows_v, gsem, wsem):
        wid = lax.axis_index("s") * _NC + lax.axis_index("c")
        base_b = wid * bpw
        pltpu.sync_copy(src_hbm.at[wid], idx_v)

        def fire_writebacks(c, b):
            bb = base_b + c * _CB
            for j in range(_CB):
                pltpu.async_copy(
                    rows_v.at[b, pl.ds(j * _L, _L)],
                    out_hbm.at[bb + j],
                    wsem.at[b],
                )

        def wait_writebacks(c, b):
            bb = base_b + c * _CB
            for j in range(_CB):
                pltpu.make_async_copy(
                    rows_v.at[b, pl.ds(j * _L, _L)],
                    out_hbm.at[bb + j],
                    wsem.at[b],
                ).wait()

        def wait_gather(c, b):
            pltpu.make_async_copy(
                table_hbm.at[idx_v.at[c]], rows_v.at[b], gsem.at[b]
            ).wait()

        for b in range(_NB):
            pltpu.async_copy(
                table_hbm.at[idx_v.at[b]], rows_v.at[b], gsem.at[b]
            )

        def group(o, carry):
            for b in range(_NB):
                wait_gather(o * _NB + b, b)
                fire_writebacks(o * _NB + b, b)
            for b in range(_NB):
                wait_writebacks(o * _NB + b, b)
                pltpu.async_copy(
                    table_hbm.at[idx_v.at[(o + 1) * _NB + b]],
                    rows_v.at[b],
                    gsem.at[b],
                )
            return carry

        lax.fori_loop(0, ngrp - 1, group, 0)

        for b in range(_NB):
            wait_gather((ngrp - 1) * _NB + b, b)
            fire_writebacks((ngrp - 1) * _NB + b, b)
        for b in range(_NB):
            wait_writebacks((ngrp - 1) * _NB + b, b)

    return gather_kernel


def kernel(source, table):
    nbatch = _B // _K
    nchunk = nbatch * _L // (_NW * _CHUNK)
    src = source.reshape(_K, _NW, nchunk, _CHUNK).astype(jnp.int32)
    gk = _make_gather(nbatch)
    parts = [gk(src[k], table) for k in range(_K)]
    return jnp.concatenate(parts, axis=0)
